# baseline (speedup 1.0000x reference)
import functools

import jax
import jax.numpy as jnp
from jax import lax
from jax.experimental import pallas as pl
from jax.experimental.pallas import tpu as pltpu
from jax.experimental.pallas import tpu_sc as plsc

NUM_CORES = 2
NUM_SUBCORES = 16
NW = NUM_CORES * NUM_SUBCORES
CHUNK = 128


def _perm(v, idx):
  dnums = lax.GatherDimensionNumbers(
      offset_dims=(), collapsed_slice_dims=(0,), start_index_map=(0,))
  return lax.gather(v, idx[:, None], dnums, (1,),
                    mode=lax.GatherScatterMode.PROMISE_IN_BOUNDS)


def _glove_body(bpw, nch, d, i_hbm, j_hbm, w_hbm, u_hbm, out_hbm,
                idx_i, idx_j, w_rows, u_rows, out_v, sem_w, sem_u):
  c = lax.axis_index("c")
  s = lax.axis_index("s")
  wid = s * NUM_CORES + c
  pltpu.sync_copy(i_hbm.at[pl.ds(wid * nch, nch)], idx_i)
  pltpu.sync_copy(j_hbm.at[pl.ds(wid * nch, nch)], idx_j)
  copies = []
  for ch in range(nch):
    copies.append(
        pltpu.async_copy(w_hbm.at[idx_i.at[ch]],
                         w_rows.at[pl.ds(ch * CHUNK, CHUNK)], sem_w))
    copies.append(
        pltpu.async_copy(u_hbm.at[idx_j.at[ch]],
                         u_rows.at[pl.ds(ch * CHUNK, CHUNK)], sem_u))
  for cp in copies:
    cp.wait()

  half = d // 2
  lane = lax.iota(jnp.int32, half)

  def group_body(g, carry):
    res = jnp.zeros((half,), jnp.float32)
    for r in range(16):
      b = g * 16 + r
      p = (w_rows[b, pl.ds(0, half)] * u_rows[b, pl.ds(0, half)] +
           w_rows[b, pl.ds(half, half)] * u_rows[b, pl.ds(half, half)])
      for sh in (8, 4, 2, 1):
        p = p + _perm(p, lane ^ sh)
      res = jnp.where(lane == r, p, res)
    out_v[pl.ds(g * 16, 16)] = res
    return carry

  lax.fori_loop(0, bpw // 16, group_body, 0)
  pltpu.sync_copy(out_v, out_hbm.at[pl.ds(wid * bpw, bpw)])


def kernel(i, j, W, U):
  b = i.shape[0]
  d = W.shape[1]
  bpw = b // NW
  nch = bpw // CHUNK
  i2 = i.reshape(NW * nch, CHUNK)
  j2 = j.reshape(NW * nch, CHUNK)

  mesh = plsc.VectorSubcoreMesh(core_axis_name="c", subcore_axis_name="s")
  run = pl.kernel(
      functools.partial(_glove_body, bpw, nch, d),
      out_type=jax.ShapeDtypeStruct((b,), jnp.float32),
      mesh=mesh,
      compiler_params=pltpu.CompilerParams(use_tc_tiling_on_sc=False),
      scratch_types=[
          pltpu.VMEM((nch, CHUNK), jnp.int32),
          pltpu.VMEM((nch, CHUNK), jnp.int32),
          pltpu.VMEM((bpw, d), jnp.float32),
          pltpu.VMEM((bpw, d), jnp.float32),
          pltpu.VMEM((bpw,), jnp.float32),
          pltpu.SemaphoreType.DMA,
          pltpu.SemaphoreType.DMA,
      ],
  )
  return run(i2, j2, W, U)


# speedup vs baseline: 1.2521x; 1.2521x over previous
import functools

import jax
import jax.numpy as jnp
from jax import lax
from jax.experimental import pallas as pl
from jax.experimental.pallas import tpu as pltpu
from jax.experimental.pallas import tpu_sc as plsc

NUM_CORES = 2
NUM_SUBCORES = 16
NW = NUM_CORES * NUM_SUBCORES
CHUNK = 128
KCOLS = 61


def _detile_body(v, ncols, wt_hbm, ut_hbm, w_out, u_out,
                 sb0, sb1, sem_r, sem_w):
  c = lax.axis_index("c")
  s = lax.axis_index("s")
  wid = s * NUM_CORES + c
  cpw = ncols // NW
  rem = ncols - cpw * NW
  tail = v - ncols * 128
  nsteps = (cpw + KCOLS - 1) // KCOLS

  def run_job(src3, dst, q, r0, width):
    pltpu.async_copy(src3.at[q, :, pl.ds(r0, width)],
                     sb0.at[:, pl.ds(0, width)], sem_r).wait()

    def copy16(o, carry):
      for dr in range(8):
        sb1[pl.ds(dr * width + o * 16, 16)] = sb0[dr, pl.ds(o * 16, 16)]
      return carry

    lax.fori_loop(0, width // 16, copy16, 0)
    writes = [
        pltpu.async_copy(sb1.at[pl.ds(dr * width, width)],
                         dst.at[pl.ds((q * 8 + dr) * v + r0, width)], sem_w)
        for dr in range(8)
    ]
    for cp in writes:
      cp.wait()

  for src3, dst in ((wt_hbm, w_out), (ut_hbm, u_out)):
    def step_body(m, carry, src3=src3, dst=dst):
      st = m // 4
      q = m % 4
      r0 = pl.multiple_of((wid * cpw + st * KCOLS) * 128, 128)
      run_job(src3, dst, q, r0, KCOLS * 128)
      return carry

    lax.fori_loop(0, nsteps * 4, step_body, 0)

  for src3, dst in ((wt_hbm, w_out), (ut_hbm, u_out)):
    if rem:
      @pl.when(wid < rem)
      def _(src3=src3, dst=dst):
        r0 = pl.multiple_of((NW * cpw + wid) * 128, 128)
        for q in range(4):
          run_job(src3, dst, q, r0, 128)
    if tail:
      @pl.when(wid == rem)
      def _(src3=src3, dst=dst):
        r0 = pl.multiple_of(ncols * 128, 128)
        for q in range(4):
          run_job(src3, dst, q, r0, tail)


def _dot_body(v, bpw, nch, i_hbm, j_hbm, wf_hbm, uf_hbm, out_hbm,
              idx_i, idx_j, idx_d, wcol, ucol, out_v, sem_w, sem_u):
  c = lax.axis_index("c")
  s = lax.axis_index("s")
  wid = s * NUM_CORES + c
  pltpu.sync_copy(i_hbm.at[pl.ds(wid * nch, nch)], idx_i)
  pltpu.sync_copy(j_hbm.at[pl.ds(wid * nch, nch)], idx_j)

  nv = CHUNK // 16

  def chunk_body(ch, carry):
    for d in range(32):
      for k in range(nv):
        sl = pl.ds(k * 16, 16)
        idx_d[0, d, sl] = idx_i[ch, sl] + d * v
        idx_d[1, d, sl] = idx_j[ch, sl] + d * v
    copies = []
    for d in range(32):
      copies.append(pltpu.async_copy(wf_hbm.at[idx_d.at[0, d]],
                                     wcol.at[d], sem_w))
      copies.append(pltpu.async_copy(uf_hbm.at[idx_d.at[1, d]],
                                     ucol.at[d], sem_u))
    for cp in copies:
      cp.wait()
    for k in range(nv):
      sl = pl.ds(k * 16, 16)
      acc = wcol[0, sl] * ucol[0, sl]
      for d in range(1, 32):
        acc = acc + wcol[d, sl] * ucol[d, sl]
      out_v[pl.ds(ch * CHUNK + k * 16, 16)] = acc
    return carry

  lax.fori_loop(0, nch, chunk_body, 0)
  pltpu.sync_copy(out_v, out_hbm.at[pl.ds(wid * bpw, bpw)])


def kernel(i, j, W, U):
  b = i.shape[0]
  v = W.shape[0]
  bpw = b // NW
  nch = bpw // CHUNK
  ncols = v // 128
  i2 = i.reshape(NW * nch, CHUNK)
  j2 = j.reshape(NW * nch, CHUNK)
  wt3 = jnp.transpose(W).reshape(4, 8, v)
  ut3 = jnp.transpose(U).reshape(4, 8, v)

  mesh = plsc.VectorSubcoreMesh(core_axis_name="c", subcore_axis_name="s")
  detile = pl.kernel(
      functools.partial(_detile_body, v, ncols),
      out_type=(jax.ShapeDtypeStruct((32 * v,), jnp.float32),
                jax.ShapeDtypeStruct((32 * v,), jnp.float32)),
      mesh=mesh,
      compiler_params=pltpu.CompilerParams(use_tc_tiling_on_sc=True),
      scratch_types=[
          pltpu.VMEM((8, KCOLS * 128), jnp.float32),
          pltpu.VMEM((8 * KCOLS * 128,), jnp.float32),
          pltpu.SemaphoreType.DMA,
          pltpu.SemaphoreType.DMA,
      ],
  )
  w_flat, u_flat = detile(wt3, ut3)

  dot = pl.kernel(
      functools.partial(_dot_body, v, bpw, nch),
      out_type=jax.ShapeDtypeStruct((b,), jnp.float32),
      mesh=mesh,
      compiler_params=pltpu.CompilerParams(use_tc_tiling_on_sc=False),
      scratch_types=[
          pltpu.VMEM((nch, CHUNK), jnp.int32),
          pltpu.VMEM((nch, CHUNK), jnp.int32),
          pltpu.VMEM((2, 32, CHUNK), jnp.int32),
          pltpu.VMEM((32, CHUNK), jnp.float32),
          pltpu.VMEM((32, CHUNK), jnp.float32),
          pltpu.VMEM((bpw,), jnp.float32),
          pltpu.SemaphoreType.DMA,
          pltpu.SemaphoreType.DMA,
      ],
  )
  return dot(i2, j2, w_flat, u_flat)


# speedup vs baseline: 1.3312x; 1.0632x over previous
import functools

import jax
import jax.numpy as jnp
from jax import lax
from jax.experimental import pallas as pl
from jax.experimental.pallas import tpu as pltpu
from jax.experimental.pallas import tpu_sc as plsc

NUM_CORES = 2
NUM_SUBCORES = 16
NW = NUM_CORES * NUM_SUBCORES
CHUNK = 128
KCOLS = 61


def _detile_body(v, ncols, wt_hbm, ut_hbm, w_out, u_out,
                 sb0, sb1, sem_r, sem_w):
  c = lax.axis_index("c")
  s = lax.axis_index("s")
  wid = s * NUM_CORES + c
  cpw = ncols // NW
  rem = ncols - cpw * NW
  tail = v - ncols * 128
  nsteps = (cpw + KCOLS - 1) // KCOLS

  def run_job(src3, dst, q, r0, width):
    pltpu.async_copy(src3.at[q, :, pl.ds(r0, width)],
                     sb0.at[:, pl.ds(0, width)], sem_r).wait()

    def copy16(o, carry):
      for dr in range(8):
        sb1[pl.ds(dr * width + o * 16, 16)] = sb0[dr, pl.ds(o * 16, 16)]
      return carry

    lax.fori_loop(0, width // 16, copy16, 0)
    writes = [
        pltpu.async_copy(sb1.at[pl.ds(dr * width, width)],
                         dst.at[pl.ds((q * 8 + dr) * v + r0, width)], sem_w)
        for dr in range(8)
    ]
    for cp in writes:
      cp.wait()

  width = KCOLS * 128

  def qr(m):
    st = m // 4
    q = m % 4
    r0 = pl.multiple_of((wid * cpw + st * KCOLS) * 128, 128)
    return q, r0

  for src3, dst in ((wt_hbm, w_out), (ut_hbm, u_out)):

    def write_descs(m, dst=dst):
      q, r0 = qr(m)
      return [
          pltpu.make_async_copy(
              sb1.at[pl.ds(dr * width, width)],
              dst.at[pl.ds((q * 8 + dr) * v + r0, width)], sem_w)
          for dr in range(8)
      ]

    def step_body(m, carry, src3=src3, dst=dst):
      q, r0 = qr(m)
      pltpu.async_copy(src3.at[q, :, pl.ds(r0, width)],
                       sb0.at[:, pl.ds(0, width)], sem_r).wait()

      @pl.when(m >= 1)
      def _():
        for cp in write_descs(m - 1):
          cp.wait()

      def copy16(o, c2):
        for dr in range(8):
          sb1[pl.ds(dr * width + o * 16, 16)] = sb0[dr, pl.ds(o * 16, 16)]
        return c2

      lax.fori_loop(0, width // 16, copy16, 0)
      for cp in write_descs(m):
        cp.start()
      return carry

    lax.fori_loop(0, nsteps * 4, step_body, 0)
    for cp in write_descs(nsteps * 4 - 1):
      cp.wait()

  for src3, dst in ((wt_hbm, w_out), (ut_hbm, u_out)):
    if rem:
      @pl.when(wid < rem)
      def _(src3=src3, dst=dst):
        r0 = pl.multiple_of((NW * cpw + wid) * 128, 128)
        for q in range(4):
          run_job(src3, dst, q, r0, 128)
    if tail:
      @pl.when(wid == rem)
      def _(src3=src3, dst=dst):
        r0 = pl.multiple_of(ncols * 128, 128)
        for q in range(4):
          run_job(src3, dst, q, r0, tail)


def _dot_body(v, bpw, nch, i_hbm, j_hbm, wf_hbm, uf_hbm, out_hbm,
              idx_i, idx_j, idx_d, wcol, ucol, out_v, sem_w, sem_u):
  c = lax.axis_index("c")
  s = lax.axis_index("s")
  wid = s * NUM_CORES + c
  pltpu.sync_copy(i_hbm.at[pl.ds(wid * nch, nch)], idx_i)
  pltpu.sync_copy(j_hbm.at[pl.ds(wid * nch, nch)], idx_j)

  nv = CHUNK // 16

  def chunk_body(ch, carry):
    for d in range(32):
      for k in range(nv):
        sl = pl.ds(k * 16, 16)
        idx_d[0, d, sl] = idx_i[ch, sl] + d * v
        idx_d[1, d, sl] = idx_j[ch, sl] + d * v
    copies = []
    for d in range(32):
      copies.append(pltpu.async_copy(wf_hbm.at[idx_d.at[0, d]],
                                     wcol.at[d], sem_w))
      copies.append(pltpu.async_copy(uf_hbm.at[idx_d.at[1, d]],
                                     ucol.at[d], sem_u))
    for cp in copies:
      cp.wait()
    for k in range(nv):
      sl = pl.ds(k * 16, 16)
      acc = wcol[0, sl] * ucol[0, sl]
      for d in range(1, 32):
        acc = acc + wcol[d, sl] * ucol[d, sl]
      out_v[pl.ds(ch * CHUNK + k * 16, 16)] = acc
    return carry

  lax.fori_loop(0, nch, chunk_body, 0)
  pltpu.sync_copy(out_v, out_hbm.at[pl.ds(wid * bpw, bpw)])


def kernel(i, j, W, U):
  b = i.shape[0]
  v = W.shape[0]
  bpw = b // NW
  nch = bpw // CHUNK
  ncols = v // 128
  i2 = i.reshape(NW * nch, CHUNK)
  j2 = j.reshape(NW * nch, CHUNK)
  wt3 = jnp.transpose(W).reshape(4, 8, v)
  ut3 = jnp.transpose(U).reshape(4, 8, v)

  mesh = plsc.VectorSubcoreMesh(core_axis_name="c", subcore_axis_name="s")
  detile = pl.kernel(
      functools.partial(_detile_body, v, ncols),
      out_type=(jax.ShapeDtypeStruct((32 * v,), jnp.float32),
                jax.ShapeDtypeStruct((32 * v,), jnp.float32)),
      mesh=mesh,
      compiler_params=pltpu.CompilerParams(use_tc_tiling_on_sc=True),
      scratch_types=[
          pltpu.VMEM((8, KCOLS * 128), jnp.float32),
          pltpu.VMEM((8 * KCOLS * 128,), jnp.float32),
          pltpu.SemaphoreType.DMA,
          pltpu.SemaphoreType.DMA,
      ],
  )
  w_flat, u_flat = detile(wt3, ut3)

  dot = pl.kernel(
      functools.partial(_dot_body, v, bpw, nch),
      out_type=jax.ShapeDtypeStruct((b,), jnp.float32),
      mesh=mesh,
      compiler_params=pltpu.CompilerParams(use_tc_tiling_on_sc=False),
      scratch_types=[
          pltpu.VMEM((nch, CHUNK), jnp.int32),
          pltpu.VMEM((nch, CHUNK), jnp.int32),
          pltpu.VMEM((2, 32, CHUNK), jnp.int32),
          pltpu.VMEM((32, CHUNK), jnp.float32),
          pltpu.VMEM((32, CHUNK), jnp.float32),
          pltpu.VMEM((bpw,), jnp.float32),
          pltpu.SemaphoreType.DMA,
          pltpu.SemaphoreType.DMA,
      ],
  )
  return dot(i2, j2, w_flat, u_flat)
